# speedup vs baseline: 1.4578x; 1.4578x over previous
import functools

import jax
import jax.numpy as jnp
from jax import lax
from jax.experimental import pallas as pl
from jax.experimental.pallas import tpu as pltpu
from jax.experimental.pallas import tpu_sc as plsc

_B = 16384
_V = 1000
_NC = 2
_NS = 16
_NW = _NC * _NS
_BPW = _B // _NW
_CHUNK = 128
_NCHUNK = _BPW // _CHUNK


def _sc_gather_body(pred_hbm, label_hbm, out_hbm, label_v, idx_v, vals_v, sem):
    wid = lax.axis_index("s") * _NC + lax.axis_index("c")
    base = wid * _BPW
    pltpu.sync_copy(label_hbm.at[pl.ds(base, _BPW)], label_v)

    def chunk(j, carry):
        rows = lax.iota(jnp.int32, 16) + (base + j * 16)
        lbl = label_v[pl.ds(j * 16, 16)]
        idx_v[pl.ds(j * 16, 16)] = rows * _V + lbl
        return carry

    lax.fori_loop(0, _BPW // 16, chunk, 0)

    copies = [
        pltpu.async_copy(
            pred_hbm.at[idx_v.at[pl.ds(k * _CHUNK, _CHUNK)]],
            vals_v.at[pl.ds(k * _CHUNK, _CHUNK)],
            sem,
        )
        for k in range(_NCHUNK)
    ]
    for c in copies:
        c.wait()
    pltpu.sync_copy(vals_v, out_hbm.at[pl.ds(base, _BPW)])


_sc_gather = functools.partial(
    pl.kernel,
    out_type=jax.ShapeDtypeStruct((_B,), jnp.float32),
    mesh=plsc.VectorSubcoreMesh(core_axis_name="c", subcore_axis_name="s"),
    scratch_types=[
        pltpu.VMEM((_BPW,), jnp.int32),
        pltpu.VMEM((_BPW,), jnp.int32),
        pltpu.VMEM((_BPW,), jnp.float32),
        pltpu.SemaphoreType.DMA,
    ],
)(_sc_gather_body)


def _tc_loss_body(vals_ref, out_ref):
    out_ref[0, 0] = -jnp.sum(jnp.log(vals_ref[...]))


_tc_loss = pl.pallas_call(
    _tc_loss_body,
    out_shape=jax.ShapeDtypeStruct((1, 1), jnp.float32),
    in_specs=[pl.BlockSpec(memory_space=pltpu.VMEM)],
    out_specs=pl.BlockSpec(memory_space=pltpu.SMEM),
)


def kernel(softMaxedPred, label):
    pred_flat = softMaxedPred.reshape(_B * _V)
    gathered = _sc_gather(pred_flat, label)
    loss = _tc_loss(gathered.reshape(128, 128))
    return loss[0, 0]


# speedup vs baseline: 1.8321x; 1.2568x over previous
import functools

import jax
import jax.numpy as jnp
from jax import lax
from jax.experimental import pallas as pl
from jax.experimental.pallas import tpu as pltpu
from jax.experimental.pallas import tpu_sc as plsc

_B = 16384
_V = 1000
_NC = 2
_NS = 16
_NW = _NC * _NS
_BPW = _B // _NW
_L = 16
_NPAIR = _BPW // _L


def _sc_gather_body(pred_hbm, label_hbm, out_hbm, label_v, blocks_v, out_v, sem):
    wid = lax.axis_index("s") * _NC + lax.axis_index("c")
    base = wid * _BPW
    pltpu.sync_copy(label_hbm.at[pl.ds(base, _BPW)], label_v)

    def pair(p, carry):
        slot = (p % 2) * 2
        row0 = pl.multiple_of(base + p * _L, 8)
        row1 = pl.multiple_of(row0 + 8, 8)
        cp0 = pltpu.make_async_copy(
            pred_hbm.at[pl.ds(row0, 8)], blocks_v.at[slot], sem)
        cp1 = pltpu.make_async_copy(
            pred_hbm.at[pl.ds(row1, 8)], blocks_v.at[slot + 1], sem)
        cp0.start()
        cp1.start()
        cp0.wait()
        cp1.wait()
        lane = lax.iota(jnp.int32, _L)
        slot_idx = slot + lax.select(
            lane >= 8, jnp.ones_like(lane), jnp.zeros_like(lane))
        r_idx = lane & 7
        c_idx = label_v[pl.ds(p * _L, _L)]
        out_v[pl.ds(p * _L, _L)] = plsc.load_gather(
            blocks_v, [slot_idx, r_idx, c_idx])
        return carry

    lax.fori_loop(0, _NPAIR, pair, 0)
    pltpu.sync_copy(out_v, out_hbm.at[pl.ds(base, _BPW)])


_sc_gather = functools.partial(
    pl.kernel,
    out_type=jax.ShapeDtypeStruct((_B,), jnp.float32),
    mesh=plsc.VectorSubcoreMesh(core_axis_name="c", subcore_axis_name="s"),
    scratch_types=[
        pltpu.VMEM((_BPW,), jnp.int32),
        pltpu.VMEM((4, 8, _V), jnp.float32),
        pltpu.VMEM((_BPW,), jnp.float32),
        pltpu.SemaphoreType.DMA,
    ],
    compiler_params=pltpu.CompilerParams(
        use_tc_tiling_on_sc=True, needs_layout_passes=False),
)(_sc_gather_body)


def _tc_loss_body(vals_ref, out_ref):
    out_ref[0, 0] = -jnp.sum(jnp.log(vals_ref[...]))


_tc_loss = pl.pallas_call(
    _tc_loss_body,
    out_shape=jax.ShapeDtypeStruct((1, 1), jnp.float32),
    in_specs=[pl.BlockSpec(memory_space=pltpu.VMEM)],
    out_specs=pl.BlockSpec(memory_space=pltpu.SMEM),
)


def kernel(softMaxedPred, label):
    gathered = _sc_gather(softMaxedPred, label)
    loss = _tc_loss(gathered.reshape(128, 128))
    return loss[0, 0]


# speedup vs baseline: 1.9972x; 1.0901x over previous
import functools

import jax
import jax.numpy as jnp
from jax import lax
from jax.experimental import pallas as pl
from jax.experimental.pallas import tpu as pltpu
from jax.experimental.pallas import tpu_sc as plsc

_B = 16384
_V = 1000
_NC = 2
_NS = 16
_NW = _NC * _NS
_BPW = _B // _NW
_L = 16
_NPAIR = _BPW // _L


def _sc_gather_body(pred_hbm, label_hbm, out_hbm, label_v, blocks_v, out_v,
                    sem0, sem1):
    wid = lax.axis_index("s") * _NC + lax.axis_index("c")
    base = wid * _BPW
    pltpu.sync_copy(label_hbm.at[pl.ds(base, _BPW)], label_v)
    sems = (sem0, sem1)

    def copies(p, parity):
        slot = parity * 2
        row0 = pl.multiple_of(base + p * _L, 8)
        row1 = pl.multiple_of(row0 + 8, 8)
        sem = sems[0] if parity == 0 else sems[1]
        return (
            pltpu.make_async_copy(
                pred_hbm.at[pl.ds(row0, 8)], blocks_v.at[slot], sem),
            pltpu.make_async_copy(
                pred_hbm.at[pl.ds(row1, 8)], blocks_v.at[slot + 1], sem),
        )

    def start(p, parity):
        c0, c1 = copies(p, parity)
        c0.start()
        c1.start()

    start(0, 0)

    def pair(i, carry):
        for parity in (0, 1):
            p = i * 2 + parity

            @pl.when(p + 1 < _NPAIR)
            def _():
                start(p + 1, 1 - parity)

            c0, c1 = copies(p, parity)
            c0.wait()
            c1.wait()
            slot = parity * 2
            lane = lax.iota(jnp.int32, _L)
            slot_idx = slot + lax.select(
                lane >= 8, jnp.ones_like(lane), jnp.zeros_like(lane))
            r_idx = lane & 7
            c_idx = label_v[pl.ds(p * _L, _L)]
            out_v[pl.ds(p * _L, _L)] = plsc.load_gather(
                blocks_v, [slot_idx, r_idx, c_idx])
        return carry

    lax.fori_loop(0, _NPAIR // 2, pair, 0)
    pltpu.sync_copy(out_v, out_hbm.at[pl.ds(base, _BPW)])


_sc_gather = functools.partial(
    pl.kernel,
    out_type=jax.ShapeDtypeStruct((_B,), jnp.float32),
    mesh=plsc.VectorSubcoreMesh(core_axis_name="c", subcore_axis_name="s"),
    scratch_types=[
        pltpu.VMEM((_BPW,), jnp.int32),
        pltpu.VMEM((4, 8, _V), jnp.float32),
        pltpu.VMEM((_BPW,), jnp.float32),
        pltpu.SemaphoreType.DMA,
        pltpu.SemaphoreType.DMA,
    ],
    compiler_params=pltpu.CompilerParams(
        use_tc_tiling_on_sc=True, needs_layout_passes=False),
)(_sc_gather_body)


def _tc_loss_body(vals_ref, out_ref):
    out_ref[0, 0] = -jnp.sum(jnp.log(vals_ref[...]))


_tc_loss = pl.pallas_call(
    _tc_loss_body,
    out_shape=jax.ShapeDtypeStruct((1, 1), jnp.float32),
    in_specs=[pl.BlockSpec(memory_space=pltpu.VMEM)],
    out_specs=pl.BlockSpec(memory_space=pltpu.SMEM),
)


def kernel(softMaxedPred, label):
    gathered = _sc_gather(softMaxedPred, label)
    loss = _tc_loss(gathered.reshape(128, 128))
    return loss[0, 0]


# speedup vs baseline: 2.0168x; 1.0098x over previous
import functools

import jax
import jax.numpy as jnp
from jax import lax
from jax.experimental import pallas as pl
from jax.experimental.pallas import tpu as pltpu
from jax.experimental.pallas import tpu_sc as plsc

_B = 16384
_V = 1000
_NC = 2
_NS = 16
_NW = _NC * _NS
_BPW = _B // _NW
_L = 16
_NPAIR = _BPW // _L


def _sc_gather_body(pred_hbm, label_hbm, out_hbm, label_v, blocks_v, out_v,
                    sem0, sem1):
    wid = lax.axis_index("s") * _NC + lax.axis_index("c")
    base = wid * _BPW
    pltpu.sync_copy(label_hbm.at[pl.ds(base, _BPW)], label_v)
    sems = (sem0, sem1)

    def copies(p, parity):
        slot = parity * 2
        row0 = pl.multiple_of(base + p * _L, 8)
        row1 = pl.multiple_of(row0 + 8, 8)
        sem = sems[0] if parity == 0 else sems[1]
        return (
            pltpu.make_async_copy(
                pred_hbm.at[pl.ds(row0, 8)], blocks_v.at[slot], sem),
            pltpu.make_async_copy(
                pred_hbm.at[pl.ds(row1, 8)], blocks_v.at[slot + 1], sem),
        )

    def start(p, parity):
        c0, c1 = copies(p, parity)
        c0.start()
        c1.start()

    start(0, 0)

    def pair(i, carry):
        for parity in (0, 1):
            p = i * 2 + parity

            @pl.when(p + 1 < _NPAIR)
            def _():
                start(p + 1, 1 - parity)

            c0, c1 = copies(p, parity)
            c0.wait()
            c1.wait()
            slot = parity * 2
            lane = lax.iota(jnp.int32, _L)
            slot_idx = slot + lax.select(
                lane >= 8, jnp.ones_like(lane), jnp.zeros_like(lane))
            r_idx = lane & 7
            c_idx = label_v[pl.ds(p * _L, _L)]
            out_v[pl.ds(p * _L, _L)] = plsc.load_gather(
                blocks_v, [slot_idx, r_idx, c_idx])
        return carry

    lax.fori_loop(0, _NPAIR // 2, pair, 0)
    pltpu.sync_copy(out_v, out_hbm.at[pl.ds(base, _BPW)])


_sc_gather = functools.partial(
    pl.kernel,
    out_type=jax.ShapeDtypeStruct((_B,), jnp.float32),
    mesh=plsc.VectorSubcoreMesh(core_axis_name="c", subcore_axis_name="s"),
    scratch_types=[
        pltpu.VMEM((_BPW,), jnp.int32),
        pltpu.VMEM((4, 8, _V), jnp.float32),
        pltpu.VMEM((_BPW,), jnp.float32),
        pltpu.SemaphoreType.DMA,
        pltpu.SemaphoreType.DMA,
    ],
    compiler_params=pltpu.CompilerParams(
        use_tc_tiling_on_sc=True, needs_layout_passes=False,
        skip_device_barrier=True),
)(_sc_gather_body)


def _tc_loss_body(vals_ref, out_ref):
    out_ref[0, 0] = -jnp.sum(jnp.log(vals_ref[...]))


_tc_loss = pl.pallas_call(
    _tc_loss_body,
    out_shape=jax.ShapeDtypeStruct((1, 1), jnp.float32),
    in_specs=[pl.BlockSpec(memory_space=pltpu.VMEM)],
    out_specs=pl.BlockSpec(memory_space=pltpu.SMEM),
)


def kernel(softMaxedPred, label):
    gathered = _sc_gather(softMaxedPred, label)
    loss = _tc_loss(gathered.reshape(128, 128))
    return loss[0, 0]


# speedup vs baseline: 4.7214x; 2.3410x over previous
import functools

import jax
import jax.numpy as jnp
from jax import lax
from jax.experimental import pallas as pl
from jax.experimental.pallas import tpu as pltpu
from jax.experimental.pallas import tpu_sc as plsc

_B = 16384
_V = 1000
_NC = 2
_NS = 16
_NW = _NC * _NS
_BPW = _B // _NW
_L = 16
_CW = 128
_NSTRIP = _BPW // _CW


def _sc_gather_body(predt_hbm, label_hbm, out_hbm, label_v, strip_v, out_v,
                    sem):
    wid = lax.axis_index("s") * _NC + lax.axis_index("c")
    base = wid * _BPW
    pltpu.sync_copy(label_hbm.at[pl.ds(base, _BPW)], label_v)

    def strip(j, carry):
        cbase = pl.multiple_of(base + j * _CW, _CW)
        pltpu.make_async_copy(
            predt_hbm.at[:, pl.ds(cbase, _CW)], strip_v, sem
        ).start()
        pltpu.make_async_copy(
            predt_hbm.at[:, pl.ds(cbase, _CW)], strip_v, sem
        ).wait()

        def window(i, c):
            w = j * (_CW // _L) + i
            col = lax.iota(jnp.int32, _L) + i * _L
            lbl = label_v[pl.ds(w * _L, _L)]
            out_v[pl.ds(w * _L, _L)] = plsc.load_gather(strip_v, [lbl, col])
            return c

        lax.fori_loop(0, _CW // _L, window, 0)
        return carry

    lax.fori_loop(0, _NSTRIP, strip, 0)
    pltpu.sync_copy(out_v, out_hbm.at[pl.ds(base, _BPW)])


_sc_gather = functools.partial(
    pl.kernel,
    out_type=jax.ShapeDtypeStruct((_B,), jnp.float32),
    mesh=plsc.VectorSubcoreMesh(core_axis_name="c", subcore_axis_name="s"),
    scratch_types=[
        pltpu.VMEM((_BPW,), jnp.int32),
        pltpu.VMEM((_V, _CW), jnp.float32),
        pltpu.VMEM((_BPW,), jnp.float32),
        pltpu.SemaphoreType.DMA,
    ],
    compiler_params=pltpu.CompilerParams(
        use_tc_tiling_on_sc=True, needs_layout_passes=False),
)(_sc_gather_body)


def _tc_loss_body(vals_ref, out_ref):
    out_ref[0, 0] = -jnp.sum(jnp.log(vals_ref[...]))


_tc_loss = pl.pallas_call(
    _tc_loss_body,
    out_shape=jax.ShapeDtypeStruct((1, 1), jnp.float32),
    in_specs=[pl.BlockSpec(memory_space=pltpu.VMEM)],
    out_specs=pl.BlockSpec(memory_space=pltpu.SMEM),
)


def kernel(softMaxedPred, label):
    gathered = _sc_gather(softMaxedPred.T, label)
    loss = _tc_loss(gathered.reshape(128, 128))
    return loss[0, 0]


# speedup vs baseline: 5.0282x; 1.0650x over previous
import functools

import jax
import jax.numpy as jnp
from jax import lax
from jax.experimental import pallas as pl
from jax.experimental.pallas import tpu as pltpu
from jax.experimental.pallas import tpu_sc as plsc

_B = 16384
_V = 1000
_NC = 2
_NS = 16
_NW = _NC * _NS
_L = 16
_CW = 128

_S = 8192
_BPW = _S // _NW
_NSTRIP = _BPW // _CW

_CB = 512
_NSTEP = (_B - _S) // _CB


def _sc_gather_body(predt_hbm, label_hbm, out_hbm, label_v, strip_v, out_v,
                    sem):
    wid = lax.axis_index("s") * _NC + lax.axis_index("c")
    base = wid * _BPW
    pltpu.sync_copy(label_hbm.at[pl.ds(base, _BPW)], label_v)

    def strip(j, carry):
        cbase = pl.multiple_of(base + j * _CW, _CW)
        pltpu.make_async_copy(
            predt_hbm.at[:, pl.ds(cbase, _CW)], strip_v, sem
        ).start()
        pltpu.make_async_copy(
            predt_hbm.at[:, pl.ds(cbase, _CW)], strip_v, sem
        ).wait()

        def window(i, c):
            w = j * (_CW // _L) + i
            col = lax.iota(jnp.int32, _L) + i * _L
            lbl = label_v[pl.ds(w * _L, _L)]
            out_v[pl.ds(w * _L, _L)] = plsc.load_gather(strip_v, [lbl, col])
            return c

        lax.fori_loop(0, _CW // _L, window, 0)
        return carry

    lax.fori_loop(0, _NSTRIP, strip, 0)
    pltpu.sync_copy(out_v, out_hbm.at[pl.ds(base, _BPW)])


_sc_gather = functools.partial(
    pl.kernel,
    out_type=jax.ShapeDtypeStruct((_S,), jnp.float32),
    mesh=plsc.VectorSubcoreMesh(core_axis_name="c", subcore_axis_name="s"),
    scratch_types=[
        pltpu.VMEM((_BPW,), jnp.int32),
        pltpu.VMEM((_V, _CW), jnp.float32),
        pltpu.VMEM((_BPW,), jnp.float32),
        pltpu.SemaphoreType.DMA,
    ],
    compiler_params=pltpu.CompilerParams(
        use_tc_tiling_on_sc=True, needs_layout_passes=False),
)(_sc_gather_body)


def _tc_dense_body(lab_ref, predt_ref, out_ref, acc_ref):
    j = pl.program_id(0)
    labels = lab_ref[0, 0, :]
    rows = lax.broadcasted_iota(jnp.int32, (_V, _CB), 0)
    picked = jnp.where(rows == labels[None, :], predt_ref[...], 0.0)
    vals = jnp.sum(picked, axis=0)
    part = -jnp.sum(jnp.log(vals))

    @pl.when(j == 0)
    def _():
        acc_ref[0] = 0.0

    acc_ref[0] += part

    @pl.when(j == _NSTEP - 1)
    def _():
        out_ref[0, 0] = acc_ref[0]


_tc_dense = pl.pallas_call(
    _tc_dense_body,
    grid=(_NSTEP,),
    in_specs=[
        pl.BlockSpec((1, 1, _CB), lambda j: (_S // _CB + j, 0, 0)),
        pl.BlockSpec((_V, _CB), lambda j: (0, _S // _CB + j)),
    ],
    out_specs=pl.BlockSpec(memory_space=pltpu.SMEM),
    out_shape=jax.ShapeDtypeStruct((1, 1), jnp.float32),
    scratch_shapes=[pltpu.SMEM((1,), jnp.float32)],
)


def _tc_combine_body(vals_ref, part_ref, out_ref):
    out_ref[0, 0] = part_ref[0] - jnp.sum(jnp.log(vals_ref[...]))


_tc_combine = pl.pallas_call(
    _tc_combine_body,
    in_specs=[
        pl.BlockSpec(memory_space=pltpu.VMEM),
        pl.BlockSpec(memory_space=pltpu.SMEM),
    ],
    out_specs=pl.BlockSpec(memory_space=pltpu.SMEM),
    out_shape=jax.ShapeDtypeStruct((1, 1), jnp.float32),
)


def kernel(softMaxedPred, label):
    predt = softMaxedPred.T
    lab3 = label.reshape(_B // _CB, 1, _CB)
    sc_vals = _sc_gather(predt, label)
    dense_part = _tc_dense(lab3, predt)
    loss = _tc_combine(sc_vals.reshape(64, 128), dense_part.reshape(1))
    return loss[0, 0]


# speedup vs baseline: 5.2311x; 1.0404x over previous
import functools

import jax
import jax.numpy as jnp
from jax import lax
from jax.experimental import pallas as pl
from jax.experimental.pallas import tpu as pltpu
from jax.experimental.pallas import tpu_sc as plsc

_B = 16384
_V = 1000
_NC = 2
_NS = 16
_NW = _NC * _NS
_L = 16
_CW = 128

_S = 8192
_BPW = _S // _NW
_NSTRIP = _BPW // _CW

_CB = 2048
_NSTEP = (_B - _S) // _CB


def _sc_gather_body(predt_hbm, label_hbm, out_hbm, label_v, strip_v, out_v,
                    sem):
    wid = lax.axis_index("s") * _NC + lax.axis_index("c")
    base = wid * _BPW
    pltpu.sync_copy(label_hbm.at[pl.ds(base, _BPW)], label_v)

    def strip(j, carry):
        cbase = pl.multiple_of(base + j * _CW, _CW)
        pltpu.make_async_copy(
            predt_hbm.at[:, pl.ds(cbase, _CW)], strip_v, sem
        ).start()
        pltpu.make_async_copy(
            predt_hbm.at[:, pl.ds(cbase, _CW)], strip_v, sem
        ).wait()

        def window(i, c):
            w = j * (_CW // _L) + i
            col = lax.iota(jnp.int32, _L) + i * _L
            lbl = label_v[pl.ds(w * _L, _L)]
            out_v[pl.ds(w * _L, _L)] = plsc.load_gather(strip_v, [lbl, col])
            return c

        lax.fori_loop(0, _CW // _L, window, 0)
        return carry

    lax.fori_loop(0, _NSTRIP, strip, 0)
    pltpu.sync_copy(out_v, out_hbm.at[pl.ds(base, _BPW)])


_sc_gather = functools.partial(
    pl.kernel,
    out_type=jax.ShapeDtypeStruct((_S,), jnp.float32),
    mesh=plsc.VectorSubcoreMesh(core_axis_name="c", subcore_axis_name="s"),
    scratch_types=[
        pltpu.VMEM((_BPW,), jnp.int32),
        pltpu.VMEM((_V, _CW), jnp.float32),
        pltpu.VMEM((_BPW,), jnp.float32),
        pltpu.SemaphoreType.DMA,
    ],
    compiler_params=pltpu.CompilerParams(
        use_tc_tiling_on_sc=True, needs_layout_passes=False),
)(_sc_gather_body)


def _tc_dense_body(lab_ref, predt_ref, out_ref, acc_ref):
    j = pl.program_id(0)
    labels = lab_ref[0, 0, :]
    rows = lax.broadcasted_iota(jnp.int32, (_V, _CB), 0)
    picked = jnp.where(rows == labels[None, :], predt_ref[...], 0.0)
    vals = jnp.sum(picked, axis=0)
    part = -jnp.sum(jnp.log(vals))

    @pl.when(j == 0)
    def _():
        acc_ref[0] = 0.0

    acc_ref[0] += part

    @pl.when(j == _NSTEP - 1)
    def _():
        out_ref[0, 0] = acc_ref[0]


_tc_dense = pl.pallas_call(
    _tc_dense_body,
    grid=(_NSTEP,),
    in_specs=[
        pl.BlockSpec((1, 1, _CB), lambda j: (_S // _CB + j, 0, 0)),
        pl.BlockSpec((_V, _CB), lambda j: (0, _S // _CB + j)),
    ],
    out_specs=pl.BlockSpec(memory_space=pltpu.SMEM),
    out_shape=jax.ShapeDtypeStruct((1, 1), jnp.float32),
    scratch_shapes=[pltpu.SMEM((1,), jnp.float32)],
)


def _tc_combine_body(vals_ref, part_ref, out_ref):
    out_ref[0, 0] = part_ref[0] - jnp.sum(jnp.log(vals_ref[...]))


_tc_combine = pl.pallas_call(
    _tc_combine_body,
    in_specs=[
        pl.BlockSpec(memory_space=pltpu.VMEM),
        pl.BlockSpec(memory_space=pltpu.SMEM),
    ],
    out_specs=pl.BlockSpec(memory_space=pltpu.SMEM),
    out_shape=jax.ShapeDtypeStruct((1, 1), jnp.float32),
)


def kernel(softMaxedPred, label):
    predt = softMaxedPred.T
    lab3 = label.reshape(_B // _CB, 1, _CB)
    sc_vals = _sc_gather(predt, label)
    dense_part = _tc_dense(lab3, predt)
    loss = _tc_combine(sc_vals.reshape(64, 128), dense_part.reshape(1))
    return loss[0, 0]


# speedup vs baseline: 5.2318x; 1.0001x over previous
import functools

import jax
import jax.numpy as jnp
from jax import lax
from jax.experimental import pallas as pl
from jax.experimental.pallas import tpu as pltpu
from jax.experimental.pallas import tpu_sc as plsc

_B = 16384
_V = 1000
_NC = 2
_NS = 16
_NW = _NC * _NS
_L = 16
_CW = 128

_S = 8192
_BPW = _S // _NW
_NSTRIP = _BPW // _CW

_CB = 4096
_NSTEP = (_B - _S) // _CB


def _sc_gather_body(predt_hbm, label_hbm, out_hbm, label_v, strip_v, out_v,
                    sem):
    wid = lax.axis_index("s") * _NC + lax.axis_index("c")
    base = wid * _BPW
    pltpu.sync_copy(label_hbm.at[pl.ds(base, _BPW)], label_v)

    def strip(j, carry):
        cbase = pl.multiple_of(base + j * _CW, _CW)
        pltpu.make_async_copy(
            predt_hbm.at[:, pl.ds(cbase, _CW)], strip_v, sem
        ).start()
        pltpu.make_async_copy(
            predt_hbm.at[:, pl.ds(cbase, _CW)], strip_v, sem
        ).wait()

        def window(i, c):
            w = j * (_CW // _L) + i
            col = lax.iota(jnp.int32, _L) + i * _L
            lbl = label_v[pl.ds(w * _L, _L)]
            out_v[pl.ds(w * _L, _L)] = plsc.load_gather(strip_v, [lbl, col])
            return c

        lax.fori_loop(0, _CW // _L, window, 0)
        return carry

    lax.fori_loop(0, _NSTRIP, strip, 0)
    pltpu.sync_copy(out_v, out_hbm.at[pl.ds(base, _BPW)])


_sc_gather = functools.partial(
    pl.kernel,
    out_type=jax.ShapeDtypeStruct((_S,), jnp.float32),
    mesh=plsc.VectorSubcoreMesh(core_axis_name="c", subcore_axis_name="s"),
    scratch_types=[
        pltpu.VMEM((_BPW,), jnp.int32),
        pltpu.VMEM((_V, _CW), jnp.float32),
        pltpu.VMEM((_BPW,), jnp.float32),
        pltpu.SemaphoreType.DMA,
    ],
    compiler_params=pltpu.CompilerParams(
        use_tc_tiling_on_sc=True, needs_layout_passes=False),
)(_sc_gather_body)


def _tc_dense_body(lab_ref, predt_ref, out_ref, acc_ref):
    j = pl.program_id(0)
    labels = lab_ref[0, 0, :]
    rows = lax.broadcasted_iota(jnp.int32, (_V, _CB), 0)
    picked = jnp.where(rows == labels[None, :], predt_ref[...], 0.0)
    vals = jnp.sum(picked, axis=0)
    part = -jnp.sum(jnp.log(vals))

    @pl.when(j == 0)
    def _():
        acc_ref[0] = 0.0

    acc_ref[0] += part

    @pl.when(j == _NSTEP - 1)
    def _():
        out_ref[0, 0] = acc_ref[0]


_tc_dense = pl.pallas_call(
    _tc_dense_body,
    grid=(_NSTEP,),
    in_specs=[
        pl.BlockSpec((1, 1, _CB), lambda j: (_S // _CB + j, 0, 0)),
        pl.BlockSpec((_V, _CB), lambda j: (0, _S // _CB + j)),
    ],
    out_specs=pl.BlockSpec(memory_space=pltpu.SMEM),
    out_shape=jax.ShapeDtypeStruct((1, 1), jnp.float32),
    scratch_shapes=[pltpu.SMEM((1,), jnp.float32)],
)


def _tc_combine_body(vals_ref, part_ref, out_ref):
    out_ref[0, 0] = part_ref[0] - jnp.sum(jnp.log(vals_ref[...]))


_tc_combine = pl.pallas_call(
    _tc_combine_body,
    in_specs=[
        pl.BlockSpec(memory_space=pltpu.VMEM),
        pl.BlockSpec(memory_space=pltpu.SMEM),
    ],
    out_specs=pl.BlockSpec(memory_space=pltpu.SMEM),
    out_shape=jax.ShapeDtypeStruct((1, 1), jnp.float32),
)


def kernel(softMaxedPred, label):
    predt = softMaxedPred.T
    lab3 = label.reshape(_B // _CB, 1, _CB)
    sc_vals = _sc_gather(predt, label)
    dense_part = _tc_dense(lab3, predt)
    loss = _tc_combine(sc_vals.reshape(64, 128), dense_part.reshape(1))
    return loss[0, 0]


# speedup vs baseline: 5.2631x; 1.0060x over previous
import functools

import jax
import jax.numpy as jnp
from jax import lax
from jax.experimental import pallas as pl
from jax.experimental.pallas import tpu as pltpu
from jax.experimental.pallas import tpu_sc as plsc

_B = 16384
_V = 1000
_NC = 2
_NS = 16
_NW = _NC * _NS
_L = 16
_CW = 128

_S = 8192
_BPW = _S // _NW
_NSTRIP = _BPW // _CW

_CB = 4096
_NSTEP = (_B - _S) // _CB


def _sc_gather_body(predt_hbm, label_hbm, out_hbm, label_v, strip_v, out_v,
                    sem):
    wid = lax.axis_index("s") * _NC + lax.axis_index("c")
    base = wid * _BPW
    pltpu.sync_copy(label_hbm.at[pl.ds(base, _BPW)], label_v)

    def strip(j, carry):
        cbase = pl.multiple_of(base + j * _CW, _CW)
        pltpu.make_async_copy(
            predt_hbm.at[:, pl.ds(cbase, _CW)], strip_v, sem
        ).start()
        pltpu.make_async_copy(
            predt_hbm.at[:, pl.ds(cbase, _CW)], strip_v, sem
        ).wait()

        def window(i, c):
            w = j * (_CW // _L) + i
            col = lax.iota(jnp.int32, _L) + i * _L
            lbl = label_v[pl.ds(w * _L, _L)]
            out_v[pl.ds(w * _L, _L)] = plsc.load_gather(strip_v, [lbl, col])
            return c

        lax.fori_loop(0, _CW // _L, window, 0)
        return carry

    lax.fori_loop(0, _NSTRIP, strip, 0)
    pltpu.sync_copy(out_v, out_hbm.at[pl.ds(base, _BPW)])


_sc_gather = functools.partial(
    pl.kernel,
    out_type=jax.ShapeDtypeStruct((_S,), jnp.float32),
    mesh=plsc.VectorSubcoreMesh(core_axis_name="c", subcore_axis_name="s"),
    scratch_types=[
        pltpu.VMEM((_BPW,), jnp.int32),
        pltpu.VMEM((_V, _CW), jnp.float32),
        pltpu.VMEM((_BPW,), jnp.float32),
        pltpu.SemaphoreType.DMA,
    ],
    compiler_params=pltpu.CompilerParams(
        use_tc_tiling_on_sc=True, needs_layout_passes=False,
        disable_bounds_checks=True, disable_semaphore_checks=True),
)(_sc_gather_body)


def _tc_dense_body(lab_ref, predt_ref, out_ref, acc_ref):
    j = pl.program_id(0)
    labels = lab_ref[0, 0, :]
    rows = lax.broadcasted_iota(jnp.int32, (_V, _CB), 0)
    picked = jnp.where(rows == labels[None, :], predt_ref[...], 0.0)
    vals = jnp.sum(picked, axis=0)
    part = -jnp.sum(jnp.log(vals))

    @pl.when(j == 0)
    def _():
        acc_ref[0] = 0.0

    acc_ref[0] += part

    @pl.when(j == _NSTEP - 1)
    def _():
        out_ref[0, 0] = acc_ref[0]


_tc_dense = pl.pallas_call(
    _tc_dense_body,
    grid=(_NSTEP,),
    in_specs=[
        pl.BlockSpec((1, 1, _CB), lambda j: (_S // _CB + j, 0, 0)),
        pl.BlockSpec((_V, _CB), lambda j: (0, _S // _CB + j)),
    ],
    out_specs=pl.BlockSpec(memory_space=pltpu.SMEM),
    out_shape=jax.ShapeDtypeStruct((1, 1), jnp.float32),
    scratch_shapes=[pltpu.SMEM((1,), jnp.float32)],
)


def _tc_combine_body(vals_ref, part_ref, out_ref):
    out_ref[0, 0] = part_ref[0] - jnp.sum(jnp.log(vals_ref[...]))


_tc_combine = pl.pallas_call(
    _tc_combine_body,
    in_specs=[
        pl.BlockSpec(memory_space=pltpu.VMEM),
        pl.BlockSpec(memory_space=pltpu.SMEM),
    ],
    out_specs=pl.BlockSpec(memory_space=pltpu.SMEM),
    out_shape=jax.ShapeDtypeStruct((1, 1), jnp.float32),
)


def kernel(softMaxedPred, label):
    predt = softMaxedPred.T
    lab3 = label.reshape(_B // _CB, 1, _CB)
    sc_vals = _sc_gather(predt, label)
    dense_part = _tc_dense(lab3, predt)
    loss = _tc_combine(sc_vals.reshape(64, 128), dense_part.reshape(1))
    return loss[0, 0]
